# baseline (speedup 1.0000x reference)
import jax
import jax.numpy as jnp
from jax import lax
from jax.experimental import pallas as pl
from jax.experimental.pallas import tpu as pltpu
from jax.experimental.pallas import tpu_sc as plsc

NODE = 10000
NODEP = 10240
REL = 1000
T = 320000
D = 128
GAMMA = 3.0
NCORES = 2
NSUB = 16
EPT = T // NSUB
EB = 80
NBLK = EPT // EB
RPT = NODEP // NSUB
FB = 16
NFL = RPT // FB

LEB = 48
LG = 22
LNCH = 19
LNBLK = LG * LNCH
LEPT = LNBLK * LEB
T2 = LEPT * NSUB
EW = 4 * LEB

_mesh = plsc.VectorSubcoreMesh(core_axis_name="c", subcore_axis_name="s",
                               num_cores=NCORES, num_subcores=NSUB)
_params = pltpu.CompilerParams(needs_layout_passes=False)
_f32 = jnp.float32


def _tanh16(x):
    e = jnp.exp(x * 2.0)
    return 1.0 - 2.0 / (e + 1.0)


def _zero_fbuf(fbuf):
    def zb(r, _):
        for j in range(D // 16):
            fbuf[r, pl.ds(j * 16, 16)] = jnp.zeros((16,), _f32)
        return 0
    lax.fori_loop(0, FB, zb, 0)


def _stats_body(src, rel, rval, row2, Ea2, Eb2, nrm,
                o_cnt, o_sA, o_sB,
                Ea, Eb, nrmtab, rowv, srcv, relv, rvalv, onesv, vA, vB,
                fb, cnt_sp, sA_sp, sB_sp):
    c = lax.axis_index("c")
    s = lax.axis_index("s")

    pltpu.sync_copy(Ea2.at[pl.ds(c * REL, REL)], Ea)
    pltpu.sync_copy(Eb2.at[pl.ds(c * REL, REL)], Eb)
    pltpu.sync_copy(nrm, nrmtab)

    def zb(r, _):
        fb[pl.ds(r * 16, 16)] = jnp.zeros((16,), _f32)
        return 0
    lax.fori_loop(0, RPT // 16, zb, 0)
    for i in range(EB // 16):
        onesv[pl.ds(i * 16, 16)] = jnp.ones((16,), _f32)
    pltpu.sync_copy(fb, cnt_sp.at[pl.ds(s * RPT, RPT)])
    pltpu.sync_copy(fb, sA_sp.at[pl.ds(s * RPT, RPT)])
    pltpu.sync_copy(fb, sB_sp.at[pl.ds(s * RPT, RPT)])
    plsc.subcore_barrier()

    def blk(b, _):
        base = s * EPT + b * EB
        pltpu.sync_copy(src.at[pl.ds(base, EB)], srcv)
        pltpu.sync_copy(rel.at[pl.ds(base, EB)], relv)
        pltpu.sync_copy(rval.at[pl.ds(base, EB)], rvalv)
        pltpu.sync_copy(row2.at[pl.ds(c * T + base, EB)], rowv)

        for i in range(EB // 16):
            sl = pl.ds(i * 16, 16)
            r16 = relv[sl]
            n16 = plsc.load_gather(nrmtab, [r16])
            z16 = (rvalv[sl] * n16) > 1e-12
            a16 = plsc.load_gather(Ea, [r16])
            b16 = plsc.load_gather(Eb, [r16])
            vA[sl] = jnp.where(z16, a16, 1.0)
            vB[sl] = jnp.where(z16, b16, 1.0)
        pltpu.sync_copy(onesv, cnt_sp.at[rowv], add=True)
        pltpu.sync_copy(vA, sA_sp.at[srcv], add=True)
        pltpu.sync_copy(vB, sB_sp.at[srcv], add=True)
        return 0

    lax.fori_loop(0, NBLK, blk, 0)
    plsc.subcore_barrier()

    def flush(sp, out):
        pltpu.sync_copy(sp.at[pl.ds(s * RPT, RPT)], fb)

        def inv(r, _):
            sl = pl.ds(r * 16, 16)
            fb[sl] = 1.0 / jnp.maximum(fb[sl], 1e-12)
            return 0
        lax.fori_loop(0, RPT // 16, inv, 0)
        pltpu.sync_copy(fb, out.at[pl.ds(c * NODEP + s * RPT, RPT)])

    flush(cnt_sp, o_cnt)
    flush(sA_sp, o_sA)
    flush(sB_sp, o_sB)


def _stats(src, rel, rval, row2, Ea2, Eb2, nrm):
    out = [jax.ShapeDtypeStruct((NCORES * NODEP,), _f32)] * 3
    return pl.kernel(
        _stats_body,
        out_type=out,
        mesh=_mesh,
        scratch_types=[
            pltpu.VMEM((REL,), _f32),
            pltpu.VMEM((REL,), _f32),
            pltpu.VMEM((REL,), _f32),
            pltpu.VMEM((EB,), jnp.int32),
            pltpu.VMEM((EB,), jnp.int32),
            pltpu.VMEM((EB,), jnp.int32),
            pltpu.VMEM((EB,), _f32),
            pltpu.VMEM((EB,), _f32),
            pltpu.VMEM((EB,), _f32),
            pltpu.VMEM((EB,), _f32),
            pltpu.VMEM((RPT,), _f32),
            pltpu.VMEM_SHARED((NODEP,), _f32),
            pltpu.VMEM_SHARED((NODEP,), _f32),
            pltpu.VMEM_SHARED((NODEP,), _f32),
        ],
        name="xgat_stats",
        compiler_params=_params,
    )(src, rel, rval, row2, Ea2, Eb2, nrm)


AEB = 64
ANB = 320
ANCH = 16
AG = ANB // ANCH
AQ = AG // 4
AW = 2 * AEB
T2A = AEB * ANB * NSUB


def _avg_body(adata, tab2, invc2, o_f,
              achunk, rid0, rid1, rid2, rid3, cid0, cid1, cid2, cid3,
              xb0, xb1, xb2, xb3, fbuf, invv,
              sg0, sg1, sg2, sg3, ss0, ss1, ss2, ss3,
              acc_sp):
    c = lax.axis_index("c")
    s = lax.axis_index("s")
    coff2 = c * NODE

    rid = (rid0, rid1, rid2, rid3)
    cid = (cid0, cid1, cid2, cid3)
    xb = (xb0, xb1, xb2, xb3)
    sg = (sg0, sg1, sg2, sg3)
    ss = (ss0, ss1, ss2, ss3)

    _zero_fbuf(fbuf)

    def zk(k, _):
        pltpu.sync_copy(fbuf, acc_sp.at[pl.ds(s * RPT + k * FB, FB)])
        return 0
    lax.fori_loop(0, NFL, zk, 0)
    plsc.subcore_barrier()

    def issueg(r, boff, gb):
        @pl.when(gb >= 4)
        def _():
            pltpu.make_async_copy(xb[r], acc_sp.at[rid[r]], ss[r]).wait()
        for i in range(AEB // 16):
            sl = pl.ds(i * 16, 16)
            rid[r][sl] = achunk[pl.ds(boff + i * 16, 16)]
            cid[r][sl] = achunk[pl.ds(boff + AEB + i * 16, 16)] + coff2
        pltpu.async_copy(tab2.at[cid[r]], xb[r], sg[r])

    def fire(r):
        pltpu.make_async_copy(tab2.at[cid[r]], xb[r], sg[r]).wait()
        pltpu.async_copy(xb[r], acc_sp.at[rid[r]], ss[r], add=True)

    def chunk(ch, _):
        pltpu.sync_copy(
            adata.at[pl.ds(((c * NSUB + s) * ANB + ch * AG) * AW, AG * AW)],
            achunk)

        def quad(q, _):
            boff = q * 4 * AW
            gb = ch * AG + q * 4
            for r in range(4):
                issueg(r, boff + r * AW, gb + r)
            for r in range(4):
                fire(r)
            return 0
        lax.fori_loop(0, AQ, quad, 0)
        return 0

    lax.fori_loop(0, ANCH, chunk, 0)
    for r in range(4):
        pltpu.make_async_copy(xb[r], acc_sp.at[rid[r]], ss[r]).wait()
    plsc.subcore_barrier()

    pltpu.sync_copy(invc2.at[pl.ds(c * NODEP + s * RPT, RPT)],
                    invv.at[pl.ds(0, RPT)])

    def fl(k, _):
        r0 = s * RPT + k * FB
        pltpu.sync_copy(acc_sp.at[pl.ds(r0, FB)], fbuf)

        def rows(rr, _):
            iv16 = invv[pl.ds(k * FB + rr, 16)]
            ivs = jnp.broadcast_to(iv16[0], (16,))
            for j in range(D // 16):
                sl = pl.ds(j * 16, 16)
                fbuf[rr, sl] = _tanh16(fbuf[rr, sl] * ivs)
            return 0
        lax.fori_loop(0, FB, rows, 0)
        pltpu.sync_copy(fbuf, o_f.at[pl.ds(c * NODEP + r0, FB)])
        return 0
    lax.fori_loop(0, NFL, fl, 0)


def _avg(adata, tab2, invc2):
    return pl.kernel(
        _avg_body,
        out_type=jax.ShapeDtypeStruct((NCORES * NODEP, D), _f32),
        mesh=_mesh,
        scratch_types=(
            [pltpu.VMEM((AG * AW,), jnp.int32)]
            + [pltpu.VMEM((AEB,), jnp.int32)] * 8
            + [pltpu.VMEM((AEB, D), _f32)] * 4
            + [
                pltpu.VMEM((FB, D), _f32),
                pltpu.VMEM((RPT + 16,), _f32),
            ]
            + [pltpu.SemaphoreType.DMA] * 8
            + [pltpu.VMEM_SHARED((NODEP, D), _f32)]
        ),
        name="xgat_avg",
        compiler_params=_params,
    )(adata, tab2, invc2)


def _layer_body(edata, U, F2s, E2, nrm, invs2, o_g,
                echunk,
                xidx0, uidx0, invidx0, ssrc0, ibuf0, wv0, bv0,
                xbuf0, ubuf0, ybuf0,
                xidx1, uidx1, invidx1, ssrc1, ibuf1, wv1, bv1,
                xbuf1, ubuf1, ybuf1,
                Etab, nrmtab, fbuf,
                semx0, semu0, semi0, semy0, semx1, semu1, semi1, semy1,
                acc_sp):
    c = lax.axis_index("c")
    s = lax.axis_index("s")
    coff = c * NODEP

    xidx = (xidx0, xidx1)
    uidx = (uidx0, uidx1)
    invidx = (invidx0, invidx1)
    ssrc = (ssrc0, ssrc1)
    ibuf = (ibuf0, ibuf1)
    wv = (wv0, wv1)
    bv = (bv0, bv1)
    xbuf = (xbuf0, xbuf1)
    ubuf = (ubuf0, ubuf1)
    ybuf = (ybuf0, ybuf1)
    semx = (semx0, semx1)
    semu = (semu0, semu1)
    semi = (semi0, semi1)
    semy = (semy0, semy1)

    pltpu.sync_copy(E2.at[pl.ds(c * REL, REL)], Etab)
    pltpu.sync_copy(nrm, nrmtab)

    _zero_fbuf(fbuf)

    def zk(k, _):
        pltpu.sync_copy(fbuf, acc_sp.at[pl.ds(s * RPT + k * FB, FB)])
        return 0
    lax.fori_loop(0, NFL, zk, 0)
    plsc.subcore_barrier()

    def issue(r, boff):
        for i in range(LEB // 16):
            sl = pl.ds(i * 16, 16)
            xidx[r][sl] = echunk[pl.ds(boff + LEB + i * 16, 16)] + coff
            uidx[r][sl] = echunk[pl.ds(boff + 2 * LEB + i * 16, 16)]
            invidx[r][sl] = echunk[pl.ds(boff + i * 16, 16)] + coff
        pltpu.async_copy(F2s.at[xidx[r]], xbuf[r], semx[r])
        pltpu.async_copy(U.at[uidx[r]], ubuf[r], semu[r])
        pltpu.async_copy(invs2.at[invidx[r]], ibuf[r], semi[r])

    def compute(r, boff, gb):
        pltpu.make_async_copy(F2s.at[xidx[r]], xbuf[r], semx[r]).wait()
        pltpu.make_async_copy(U.at[uidx[r]], ubuf[r], semu[r]).wait()
        pltpu.make_async_copy(invs2.at[invidx[r]], ibuf[r], semi[r]).wait()

        for i in range(LEB // 16):
            sl = pl.ds(i * 16, 16)
            r16 = echunk[pl.ds(boff + 2 * LEB + i * 16, 16)]
            rv16 = plsc.bitcast(echunk[pl.ds(boff + 3 * LEB + i * 16, 16)],
                                _f32)
            n16 = plsc.load_gather(nrmtab, [r16])
            z16 = (rv16 * n16) > 1e-12
            e16 = plsc.load_gather(Etab, [r16])
            w16 = jnp.where(z16, e16, 1.0) * ibuf[r][sl]
            wv[r][sl] = w16
            bv[r][sl] = jnp.where(z16, 2.0 * w16, 0.0)

        @pl.when(gb >= 2)
        def _():
            pltpu.make_async_copy(ybuf[r], acc_sp.at[ssrc[r]], semy[r]).wait()

        @plsc.parallel_loop(0, LEB, step=1, unroll=2)
        def _(e):
            xs = [xbuf[r][e, pl.ds(j * 16, 16)] for j in range(D // 16)]
            us = [ubuf[r][e, pl.ds(j * 16, 16)] for j in range(D // 16)]
            acc = xs[0] * us[0]
            for j in range(1, D // 16):
                acc = acc + xs[j] * us[j]
            dsum = jnp.sum(acc)
            alpha = wv[r][pl.ds(e, 16)][0]
            beta = bv[r][pl.ds(e, 16)][0] * dsum
            for j in range(D // 16):
                ybuf[r][e, pl.ds(j * 16, 16)] = alpha * xs[j] - beta * us[j]

        for i in range(LEB // 16):
            sl = pl.ds(i * 16, 16)
            ssrc[r][sl] = echunk[pl.ds(boff + i * 16, 16)]
        pltpu.async_copy(ybuf[r], acc_sp.at[ssrc[r]], semy[r], add=True)

    def chunk(ch, _):
        pltpu.sync_copy(
            edata.at[pl.ds((s * LNBLK + ch * LG) * EW, LG * EW)], echunk)
        issue(0, 0)

        def pair(p, _):
            boff = p * 2 * EW
            gb = ch * LG + p * 2
            issue(1, boff + EW)
            compute(0, boff, gb)

            @pl.when(p < LG // 2 - 1)
            def _():
                issue(0, boff + 2 * EW)

            compute(1, boff + EW, gb + 1)
            return 0

        lax.fori_loop(0, LG // 2, pair, 0)
        return 0

    lax.fori_loop(0, LNCH, chunk, 0)
    pltpu.make_async_copy(ybuf[0], acc_sp.at[ssrc[0]], semy[0]).wait()
    pltpu.make_async_copy(ybuf[1], acc_sp.at[ssrc[1]], semy[1]).wait()
    plsc.subcore_barrier()

    def fl(k, _):
        r0 = s * RPT + k * FB
        pltpu.sync_copy(acc_sp.at[pl.ds(r0, FB)], fbuf)

        @plsc.parallel_loop(0, FB, step=1, unroll=2)
        def _(rr):
            for j in range(D // 16):
                sl = pl.ds(j * 16, 16)
                fbuf[rr, sl] = _tanh16(fbuf[rr, sl])
        pltpu.sync_copy(fbuf, o_g.at[pl.ds(c * NODEP + r0, FB)])
        return 0
    lax.fori_loop(0, NFL, fl, 0)


def _layer(edata, U, F2s, E2, nrm, invs2):
    slot = [
        pltpu.VMEM((LEB,), jnp.int32),
        pltpu.VMEM((LEB,), jnp.int32),
        pltpu.VMEM((LEB,), jnp.int32),
        pltpu.VMEM((LEB,), jnp.int32),
        pltpu.VMEM((LEB,), _f32),
        pltpu.VMEM((LEB + 16,), _f32),
        pltpu.VMEM((LEB + 16,), _f32),
        pltpu.VMEM((LEB, D), _f32),
        pltpu.VMEM((LEB, D), _f32),
        pltpu.VMEM((LEB, D), _f32),
    ]
    return pl.kernel(
        _layer_body,
        out_type=jax.ShapeDtypeStruct((NCORES * NODEP, D), _f32),
        mesh=_mesh,
        scratch_types=(
            [pltpu.VMEM((LG * EW,), jnp.int32)]
            + slot + slot
            + [
                pltpu.VMEM((REL,), _f32),
                pltpu.VMEM((REL,), _f32),
                pltpu.VMEM((FB, D), _f32),
            ]
            + [pltpu.SemaphoreType.DMA] * 8
            + [pltpu.VMEM_SHARED((NODEP, D), _f32)]
        ),
        name="xgat_layer",
        compiler_params=_params,
    )(edata, U, F2s, E2, nrm, invs2)


NB = 640
NJ = NODEP // NB
M = 512


def _loss_body(A_ref, lf_ref, rf_ref, emb_ref, out_ref,
               X, posb, sumx, sumx2, xmax, sumexp):
    p = pl.program_id(0)
    j = pl.program_id(1)

    @pl.when((p == 0) & (j == 0))
    def _():
        dif = A_ref[0:256, :] - A_ref[256:512, :]
        pv = jnp.sum(dif * dif, axis=1, keepdims=True)
        posb[0:256, :] = pv
        posb[256:512, :] = pv
        sumx[...] = jnp.zeros((M, 1), _f32)
        sumx2[...] = jnp.zeros((M, 1), _f32)
        xmax[...] = jnp.full((M, 1), -jnp.inf, _f32)
        sumexp[...] = jnp.zeros((M, 1), _f32)

    colid = (jnp.float32(1.0) * j * NB
             + lax.broadcasted_iota(jnp.int32, (M, NB), 1).astype(_f32))
    valid = colid < float(NODE)

    @pl.when(p == 0)
    def _():
        a = A_ref[...]
        nb = emb_ref[...]
        g = lax.dot_general(a, nb, (((1,), (1,)), ((), ())),
                            preferred_element_type=_f32)
        n2 = jnp.sum(nb * nb, axis=1)
        a2 = jnp.sum(a * a, axis=1, keepdims=True)
        sq = a2 + n2[None, :] - 2.0 * g
        mask = (1.0
                - (colid == lf_ref[...]).astype(_f32)
                - (colid == rf_ref[...]).astype(_f32))
        x = (posb[...] - sq + GAMMA) * mask
        x = jnp.where(valid, x, 0.0)
        X[:, pl.ds(j * NB, NB)] = x
        sumx[...] += jnp.sum(x, axis=1, keepdims=True)
        sumx2[...] += jnp.sum(x * x, axis=1, keepdims=True)
        xm = jnp.max(jnp.where(valid, x, -jnp.inf), axis=1, keepdims=True)
        xmax[...] = jnp.maximum(xmax[...], xm)

    @pl.when(p == 1)
    def _():
        m = sumx[...] * (1.0 / NODE)
        sd = jnp.sqrt(sumx2[...] * (1.0 / NODE) - m * m)
        zmax = 20.0 * (xmax[...] - m) / sd + 8.0
        x = X[:, pl.ds(j * NB, NB)]
        zz = 20.0 * (x - m) / sd + 8.0
        pe = jnp.where(valid, jnp.exp(zz - zmax), 0.0)
        sumexp[...] += jnp.sum(pe, axis=1, keepdims=True)

        @pl.when(j == NJ - 1)
        def _():
            ll = zmax + jnp.log(sumexp[...])
            out_ref[...] = jnp.reshape(jnp.sum(ll) * (1.0 / 256.0), (1, 1))


def _loss(A, lf, rf, emb):
    return pl.pallas_call(
        _loss_body,
        grid=(2, NJ),
        in_specs=[
            pl.BlockSpec((M, 6 * D), lambda p, j: (0, 0)),
            pl.BlockSpec((M, 1), lambda p, j: (0, 0)),
            pl.BlockSpec((M, 1), lambda p, j: (0, 0)),
            pl.BlockSpec((NB, 6 * D), lambda p, j: (j, 0)),
        ],
        out_specs=pl.BlockSpec((1, 1), lambda p, j: (0, 0)),
        out_shape=jax.ShapeDtypeStruct((1, 1), _f32),
        scratch_shapes=[
            pltpu.VMEM((M, NODEP), _f32),
            pltpu.VMEM((M, 1), _f32),
            pltpu.VMEM((M, 1), _f32),
            pltpu.VMEM((M, 1), _f32),
            pltpu.VMEM((M, 1), _f32),
            pltpu.VMEM((M, 1), _f32),
        ],
    )(A, lf, rf, emb)


def kernel(train_pairs, adj, r_index1, r_val, ent_adj, rel_adj_row,
           rel_adj_col, ent_table, rel_table, attn_e0, attn_e1, attn_r0,
           attn_r1):
    src = adj[0]
    dst = adj[1]
    nrm = jnp.sqrt(jnp.sum(rel_table * rel_table, axis=1))
    nrmc = jnp.maximum(nrm, 1e-30)
    U = rel_table / nrmc[:, None]

    def etab(ak):
        return jnp.exp((rel_table @ ak)[:, 0] / nrmc)

    Ea2 = jnp.concatenate([etab(attn_e0), etab(attn_r0)])
    Eb2 = jnp.concatenate([etab(attn_e1), etab(attn_r1)])

    row2 = jnp.concatenate([ent_adj[0], rel_adj_row])
    col2 = jnp.concatenate([ent_adj[1], rel_adj_col])
    tab2 = jnp.concatenate([ent_table, rel_table], axis=0)

    invc2, inv_sA2, inv_sB2 = _stats(src, r_index1, r_val, row2, Ea2, Eb2, nrm)

    pad = T2 - T
    i32 = jnp.int32
    srcp = jnp.concatenate([src, jnp.full((pad,), NODE, i32)])
    dstp = jnp.concatenate([dst, jnp.full((pad,), NODE, i32)])
    relp = jnp.concatenate([r_index1, jnp.zeros((pad,), i32)])
    rvb = lax.bitcast_convert_type(r_val, i32)
    rvbp = jnp.concatenate([rvb, jnp.full((pad,), 0x3F800000, i32)])
    edata = (jnp.stack([srcp, dstp, relp, rvbp], axis=0)
             .reshape(4, T2 // LEB, LEB)
             .transpose(1, 0, 2)
             .reshape(-1))

    padA = T2A - T
    rowp0 = jnp.concatenate([ent_adj[0], jnp.full((padA,), NODE, i32)])
    colp0 = jnp.concatenate([ent_adj[1], jnp.zeros((padA,), i32)])
    rowp1 = jnp.concatenate([rel_adj_row, jnp.full((padA,), NODE, i32)])
    colp1 = jnp.concatenate([rel_adj_col, jnp.zeros((padA,), i32)])

    def _ilv(rowp, colp):
        return (jnp.stack([rowp, colp], axis=0)
                .reshape(2, T2A // AEB, AEB)
                .transpose(1, 0, 2)
                .reshape(-1))

    adata = jnp.concatenate([_ilv(rowp0, colp0), _ilv(rowp1, colp1)])
    F0 = _avg(adata, tab2, invc2)
    F1 = _layer(edata, U, F0, Ea2, nrm, inv_sA2)
    F2 = _layer(edata, U, F1, Eb2, nrm, inv_sB2)

    emb = jnp.concatenate([F0[:NODEP], F1[:NODEP], F2[:NODEP],
                           F0[NODEP:], F1[NODEP:], F2[NODEP:]], axis=1)
    l = train_pairs[:, 0]
    r = train_pairs[:, 1]
    A = jnp.concatenate([emb[l], emb[r]], axis=0)
    lf = jnp.concatenate([l, r]).astype(_f32)[:, None]
    rf = jnp.concatenate([r, l]).astype(_f32)[:, None]
    loss = _loss(A, lf, rf, emb)
    return jnp.reshape(loss, ())


# speedup vs baseline: 1.2366x; 1.2366x over previous
import jax
import jax.numpy as jnp
from jax import lax
from jax.experimental import pallas as pl
from jax.experimental.pallas import tpu as pltpu
from jax.experimental.pallas import tpu_sc as plsc

NODE = 10000
NODEP = 10240
REL = 1000
T = 320000
D = 128
GAMMA = 3.0
NCORES = 2
NSUB = 16
EPT = T // NSUB
EB = 80
NBLK = EPT // EB
RPT = NODEP // NSUB
FB = 16
NFL = RPT // FB

LEB = 48
LG = 22
LNCH = 19
LNBLK = LG * LNCH
LEPT = LNBLK * LEB
T2 = LEPT * NSUB
EW = 4 * LEB

_mesh = plsc.VectorSubcoreMesh(core_axis_name="c", subcore_axis_name="s",
                               num_cores=NCORES, num_subcores=NSUB)
_params = pltpu.CompilerParams(needs_layout_passes=False)
_f32 = jnp.float32


def _tanh16(x):
    e = jnp.exp(x * 2.0)
    return 1.0 - 2.0 / (e + 1.0)


def _zero_fbuf(fbuf):
    def zb(r, _):
        for j in range(D // 16):
            fbuf[r, pl.ds(j * 16, 16)] = jnp.zeros((16,), _f32)
        return 0
    lax.fori_loop(0, FB, zb, 0)


SEB = 128
SNB = 160
SNCH = 20
SG = SNB // SNCH
SW = 4 * SEB
T2S = SEB * SNB * NSUB


def _stats_body(sdata, Ea2, Eb2, nrm,
                o_cnt, o_sA, o_sB,
                schunk, rb0, rb1, rb2, rb3, sb0, sb1, sb2, sb3,
                vA0, vA1, vA2, vA3, vB0, vB1, vB2, vB3, onesv,
                Ea, Eb, nrmtab, fb,
                sm0, sm1, sm2, sm3,
                cnt_sp, sA_sp, sB_sp):
    c = lax.axis_index("c")
    s = lax.axis_index("s")

    rb = (rb0, rb1, rb2, rb3)
    sb = (sb0, sb1, sb2, sb3)
    vA = (vA0, vA1, vA2, vA3)
    vB = (vB0, vB1, vB2, vB3)
    sm = (sm0, sm1, sm2, sm3)

    pltpu.sync_copy(Ea2.at[pl.ds(c * REL, REL)], Ea)
    pltpu.sync_copy(Eb2.at[pl.ds(c * REL, REL)], Eb)
    pltpu.sync_copy(nrm, nrmtab)

    def zb(r, _):
        fb[pl.ds(r * 16, 16)] = jnp.zeros((16,), _f32)
        return 0
    lax.fori_loop(0, RPT // 16, zb, 0)
    for i in range(SEB // 16):
        onesv[pl.ds(i * 16, 16)] = jnp.ones((16,), _f32)
    pltpu.sync_copy(fb, cnt_sp.at[pl.ds(s * RPT, RPT)])
    pltpu.sync_copy(fb, sA_sp.at[pl.ds(s * RPT, RPT)])
    pltpu.sync_copy(fb, sB_sp.at[pl.ds(s * RPT, RPT)])
    plsc.subcore_barrier()

    def drain(r):
        pltpu.make_async_copy(onesv, cnt_sp.at[rb[r]], sm[r]).wait()
        pltpu.make_async_copy(vA[r], sA_sp.at[sb[r]], sm[r]).wait()
        pltpu.make_async_copy(vB[r], sB_sp.at[sb[r]], sm[r]).wait()

    def block(r, boff, gb):
        @pl.when(gb >= 4)
        def _():
            drain(r)
        for i in range(SEB // 16):
            sl = pl.ds(i * 16, 16)
            rb[r][sl] = schunk[pl.ds(boff + i * 16, 16)]
            sb[r][sl] = schunk[pl.ds(boff + SEB + i * 16, 16)]
            r16 = schunk[pl.ds(boff + 2 * SEB + i * 16, 16)]
            rv16 = plsc.bitcast(schunk[pl.ds(boff + 3 * SEB + i * 16, 16)],
                                _f32)
            n16 = plsc.load_gather(nrmtab, [r16])
            z16 = (rv16 * n16) > 1e-12
            a16 = plsc.load_gather(Ea, [r16])
            b16 = plsc.load_gather(Eb, [r16])
            vA[r][sl] = jnp.where(z16, a16, 1.0)
            vB[r][sl] = jnp.where(z16, b16, 1.0)
        pltpu.async_copy(onesv, cnt_sp.at[rb[r]], sm[r], add=True)
        pltpu.async_copy(vA[r], sA_sp.at[sb[r]], sm[r], add=True)
        pltpu.async_copy(vB[r], sB_sp.at[sb[r]], sm[r], add=True)

    def chunk(ch, _):
        pltpu.sync_copy(
            sdata.at[pl.ds(((c * NSUB + s) * SNB + ch * SG) * SW, SG * SW)],
            schunk)

        def quad(q, _):
            for r in range(4):
                block(r, (q * 4 + r) * SW, ch * SG + q * 4 + r)
            return 0
        lax.fori_loop(0, SG // 4, quad, 0)
        return 0

    lax.fori_loop(0, SNCH, chunk, 0)
    for r in range(4):
        drain(r)
    plsc.subcore_barrier()

    def flush(sp, out):
        pltpu.sync_copy(sp.at[pl.ds(s * RPT, RPT)], fb)

        def inv(r, _):
            sl = pl.ds(r * 16, 16)
            fb[sl] = 1.0 / jnp.maximum(fb[sl], 1e-12)
            return 0
        lax.fori_loop(0, RPT // 16, inv, 0)
        pltpu.sync_copy(fb, out.at[pl.ds(c * NODEP + s * RPT, RPT)])

    flush(cnt_sp, o_cnt)
    flush(sA_sp, o_sA)
    flush(sB_sp, o_sB)


def _stats(sdata, Ea2, Eb2, nrm):
    out = [jax.ShapeDtypeStruct((NCORES * NODEP,), _f32)] * 3
    return pl.kernel(
        _stats_body,
        out_type=out,
        mesh=_mesh,
        scratch_types=(
            [pltpu.VMEM((SG * SW,), jnp.int32)]
            + [pltpu.VMEM((SEB,), jnp.int32)] * 8
            + [pltpu.VMEM((SEB,), _f32)] * 8
            + [
                pltpu.VMEM((SEB,), _f32),
                pltpu.VMEM((REL,), _f32),
                pltpu.VMEM((REL,), _f32),
                pltpu.VMEM((REL,), _f32),
                pltpu.VMEM((RPT,), _f32),
            ]
            + [pltpu.SemaphoreType.DMA] * 4
            + [
                pltpu.VMEM_SHARED((NODEP,), _f32),
                pltpu.VMEM_SHARED((NODEP,), _f32),
                pltpu.VMEM_SHARED((NODEP,), _f32),
            ]
        ),
        name="xgat_stats",
        compiler_params=_params,
    )(sdata, Ea2, Eb2, nrm)


AEB = 64
ANB = 320
ANCH = 16
AG = ANB // ANCH
AQ = AG // 4
AW = 2 * AEB
T2A = AEB * ANB * NSUB


def _avg_body(adata, tab2, invc2, o_f,
              achunk, rid0, rid1, rid2, rid3, cid0, cid1, cid2, cid3,
              xb0, xb1, xb2, xb3, fbuf, invv,
              sg0, sg1, sg2, sg3, ss0, ss1, ss2, ss3,
              acc_sp):
    c = lax.axis_index("c")
    s = lax.axis_index("s")
    coff2 = c * NODE

    rid = (rid0, rid1, rid2, rid3)
    cid = (cid0, cid1, cid2, cid3)
    xb = (xb0, xb1, xb2, xb3)
    sg = (sg0, sg1, sg2, sg3)
    ss = (ss0, ss1, ss2, ss3)

    _zero_fbuf(fbuf)

    def zk(k, _):
        pltpu.sync_copy(fbuf, acc_sp.at[pl.ds(s * RPT + k * FB, FB)])
        return 0
    lax.fori_loop(0, NFL, zk, 0)
    plsc.subcore_barrier()

    def issueg(r, boff, gb):
        @pl.when(gb >= 4)
        def _():
            pltpu.make_async_copy(xb[r], acc_sp.at[rid[r]], ss[r]).wait()
        for i in range(AEB // 16):
            sl = pl.ds(i * 16, 16)
            rid[r][sl] = achunk[pl.ds(boff + i * 16, 16)]
            cid[r][sl] = achunk[pl.ds(boff + AEB + i * 16, 16)] + coff2
        pltpu.async_copy(tab2.at[cid[r]], xb[r], sg[r])

    def fire(r):
        pltpu.make_async_copy(tab2.at[cid[r]], xb[r], sg[r]).wait()
        pltpu.async_copy(xb[r], acc_sp.at[rid[r]], ss[r], add=True)

    def chunk(ch, _):
        pltpu.sync_copy(
            adata.at[pl.ds(((c * NSUB + s) * ANB + ch * AG) * AW, AG * AW)],
            achunk)

        def quad(q, _):
            boff = q * 4 * AW
            gb = ch * AG + q * 4
            for r in range(4):
                issueg(r, boff + r * AW, gb + r)
            for r in range(4):
                fire(r)
            return 0
        lax.fori_loop(0, AQ, quad, 0)
        return 0

    lax.fori_loop(0, ANCH, chunk, 0)
    for r in range(4):
        pltpu.make_async_copy(xb[r], acc_sp.at[rid[r]], ss[r]).wait()
    plsc.subcore_barrier()

    pltpu.sync_copy(invc2.at[pl.ds(c * NODEP + s * RPT, RPT)],
                    invv.at[pl.ds(0, RPT)])

    def fl(k, _):
        r0 = s * RPT + k * FB
        pltpu.sync_copy(acc_sp.at[pl.ds(r0, FB)], fbuf)

        def rows(rr, _):
            iv16 = invv[pl.ds(k * FB + rr, 16)]
            ivs = jnp.broadcast_to(iv16[0], (16,))
            for j in range(D // 16):
                sl = pl.ds(j * 16, 16)
                fbuf[rr, sl] = _tanh16(fbuf[rr, sl] * ivs)
            return 0
        lax.fori_loop(0, FB, rows, 0)
        pltpu.sync_copy(fbuf, o_f.at[pl.ds(c * NODEP + r0, FB)])
        return 0
    lax.fori_loop(0, NFL, fl, 0)


def _avg(adata, tab2, invc2):
    return pl.kernel(
        _avg_body,
        out_type=jax.ShapeDtypeStruct((NCORES * NODEP, D), _f32),
        mesh=_mesh,
        scratch_types=(
            [pltpu.VMEM((AG * AW,), jnp.int32)]
            + [pltpu.VMEM((AEB,), jnp.int32)] * 8
            + [pltpu.VMEM((AEB, D), _f32)] * 4
            + [
                pltpu.VMEM((FB, D), _f32),
                pltpu.VMEM((RPT + 16,), _f32),
            ]
            + [pltpu.SemaphoreType.DMA] * 8
            + [pltpu.VMEM_SHARED((NODEP, D), _f32)]
        ),
        name="xgat_avg",
        compiler_params=_params,
    )(adata, tab2, invc2)


def _layer_body(edata, U, F2s, E2, nrm, invs2, o_g,
                echunk,
                xidx0, uidx0, invidx0, ssrc0, ibuf0, wv0, bv0,
                xbuf0, ubuf0, ybuf0,
                xidx1, uidx1, invidx1, ssrc1, ibuf1, wv1, bv1,
                xbuf1, ubuf1, ybuf1,
                Etab, nrmtab, fbuf,
                semx0, semu0, semi0, semy0, semx1, semu1, semi1, semy1,
                acc_sp):
    c = lax.axis_index("c")
    s = lax.axis_index("s")
    coff = c * NODEP

    xidx = (xidx0, xidx1)
    uidx = (uidx0, uidx1)
    invidx = (invidx0, invidx1)
    ssrc = (ssrc0, ssrc1)
    ibuf = (ibuf0, ibuf1)
    wv = (wv0, wv1)
    bv = (bv0, bv1)
    xbuf = (xbuf0, xbuf1)
    ubuf = (ubuf0, ubuf1)
    ybuf = (ybuf0, ybuf1)
    semx = (semx0, semx1)
    semu = (semu0, semu1)
    semi = (semi0, semi1)
    semy = (semy0, semy1)

    pltpu.sync_copy(E2.at[pl.ds(c * REL, REL)], Etab)
    pltpu.sync_copy(nrm, nrmtab)

    _zero_fbuf(fbuf)

    def zk(k, _):
        pltpu.sync_copy(fbuf, acc_sp.at[pl.ds(s * RPT + k * FB, FB)])
        return 0
    lax.fori_loop(0, NFL, zk, 0)
    plsc.subcore_barrier()

    def issue(r, boff):
        for i in range(LEB // 16):
            sl = pl.ds(i * 16, 16)
            xidx[r][sl] = echunk[pl.ds(boff + LEB + i * 16, 16)] + coff
            uidx[r][sl] = echunk[pl.ds(boff + 2 * LEB + i * 16, 16)]
            invidx[r][sl] = echunk[pl.ds(boff + i * 16, 16)] + coff
        pltpu.async_copy(F2s.at[xidx[r]], xbuf[r], semx[r])
        pltpu.async_copy(U.at[uidx[r]], ubuf[r], semu[r])
        pltpu.async_copy(invs2.at[invidx[r]], ibuf[r], semi[r])

    def compute(r, boff, gb):
        pltpu.make_async_copy(F2s.at[xidx[r]], xbuf[r], semx[r]).wait()
        pltpu.make_async_copy(U.at[uidx[r]], ubuf[r], semu[r]).wait()
        pltpu.make_async_copy(invs2.at[invidx[r]], ibuf[r], semi[r]).wait()

        for i in range(LEB // 16):
            sl = pl.ds(i * 16, 16)
            r16 = echunk[pl.ds(boff + 2 * LEB + i * 16, 16)]
            rv16 = plsc.bitcast(echunk[pl.ds(boff + 3 * LEB + i * 16, 16)],
                                _f32)
            n16 = plsc.load_gather(nrmtab, [r16])
            z16 = (rv16 * n16) > 1e-12
            e16 = plsc.load_gather(Etab, [r16])
            w16 = jnp.where(z16, e16, 1.0) * ibuf[r][sl]
            wv[r][sl] = w16
            bv[r][sl] = jnp.where(z16, 2.0 * w16, 0.0)

        @pl.when(gb >= 2)
        def _():
            pltpu.make_async_copy(ybuf[r], acc_sp.at[ssrc[r]], semy[r]).wait()

        @plsc.parallel_loop(0, LEB, step=1, unroll=2)
        def _(e):
            xs = [xbuf[r][e, pl.ds(j * 16, 16)] for j in range(D // 16)]
            us = [ubuf[r][e, pl.ds(j * 16, 16)] for j in range(D // 16)]
            acc = xs[0] * us[0]
            for j in range(1, D // 16):
                acc = acc + xs[j] * us[j]
            dsum = jnp.sum(acc)
            alpha = wv[r][pl.ds(e, 16)][0]
            beta = bv[r][pl.ds(e, 16)][0] * dsum
            for j in range(D // 16):
                ybuf[r][e, pl.ds(j * 16, 16)] = alpha * xs[j] - beta * us[j]

        for i in range(LEB // 16):
            sl = pl.ds(i * 16, 16)
            ssrc[r][sl] = echunk[pl.ds(boff + i * 16, 16)]
        pltpu.async_copy(ybuf[r], acc_sp.at[ssrc[r]], semy[r], add=True)

    def chunk(ch, _):
        pltpu.sync_copy(
            edata.at[pl.ds((s * LNBLK + ch * LG) * EW, LG * EW)], echunk)
        issue(0, 0)

        def pair(p, _):
            boff = p * 2 * EW
            gb = ch * LG + p * 2
            issue(1, boff + EW)
            compute(0, boff, gb)

            @pl.when(p < LG // 2 - 1)
            def _():
                issue(0, boff + 2 * EW)

            compute(1, boff + EW, gb + 1)
            return 0

        lax.fori_loop(0, LG // 2, pair, 0)
        return 0

    lax.fori_loop(0, LNCH, chunk, 0)
    pltpu.make_async_copy(ybuf[0], acc_sp.at[ssrc[0]], semy[0]).wait()
    pltpu.make_async_copy(ybuf[1], acc_sp.at[ssrc[1]], semy[1]).wait()
    plsc.subcore_barrier()

    def fl(k, _):
        r0 = s * RPT + k * FB
        pltpu.sync_copy(acc_sp.at[pl.ds(r0, FB)], fbuf)

        @plsc.parallel_loop(0, FB, step=1, unroll=2)
        def _(rr):
            for j in range(D // 16):
                sl = pl.ds(j * 16, 16)
                fbuf[rr, sl] = _tanh16(fbuf[rr, sl])
        pltpu.sync_copy(fbuf, o_g.at[pl.ds(c * NODEP + r0, FB)])
        return 0
    lax.fori_loop(0, NFL, fl, 0)


def _layer(edata, U, F2s, E2, nrm, invs2):
    slot = [
        pltpu.VMEM((LEB,), jnp.int32),
        pltpu.VMEM((LEB,), jnp.int32),
        pltpu.VMEM((LEB,), jnp.int32),
        pltpu.VMEM((LEB,), jnp.int32),
        pltpu.VMEM((LEB,), _f32),
        pltpu.VMEM((LEB + 16,), _f32),
        pltpu.VMEM((LEB + 16,), _f32),
        pltpu.VMEM((LEB, D), _f32),
        pltpu.VMEM((LEB, D), _f32),
        pltpu.VMEM((LEB, D), _f32),
    ]
    return pl.kernel(
        _layer_body,
        out_type=jax.ShapeDtypeStruct((NCORES * NODEP, D), _f32),
        mesh=_mesh,
        scratch_types=(
            [pltpu.VMEM((LG * EW,), jnp.int32)]
            + slot + slot
            + [
                pltpu.VMEM((REL,), _f32),
                pltpu.VMEM((REL,), _f32),
                pltpu.VMEM((FB, D), _f32),
            ]
            + [pltpu.SemaphoreType.DMA] * 8
            + [pltpu.VMEM_SHARED((NODEP, D), _f32)]
        ),
        name="xgat_layer",
        compiler_params=_params,
    )(edata, U, F2s, E2, nrm, invs2)


NB = 640
NJ = NODEP // NB
M = 512


def _loss_body(A_ref, lf_ref, rf_ref, emb_ref, out_ref,
               X, posb, sumx, sumx2, xmax, sumexp):
    p = pl.program_id(0)
    j = pl.program_id(1)

    @pl.when((p == 0) & (j == 0))
    def _():
        dif = A_ref[0:256, :] - A_ref[256:512, :]
        pv = jnp.sum(dif * dif, axis=1, keepdims=True)
        posb[0:256, :] = pv
        posb[256:512, :] = pv
        sumx[...] = jnp.zeros((M, 1), _f32)
        sumx2[...] = jnp.zeros((M, 1), _f32)
        xmax[...] = jnp.full((M, 1), -jnp.inf, _f32)
        sumexp[...] = jnp.zeros((M, 1), _f32)

    colid = (jnp.float32(1.0) * j * NB
             + lax.broadcasted_iota(jnp.int32, (M, NB), 1).astype(_f32))
    valid = colid < float(NODE)

    @pl.when(p == 0)
    def _():
        a = A_ref[...]
        nb = emb_ref[...]
        g = lax.dot_general(a, nb, (((1,), (1,)), ((), ())),
                            preferred_element_type=_f32)
        n2 = jnp.sum(nb * nb, axis=1)
        a2 = jnp.sum(a * a, axis=1, keepdims=True)
        sq = a2 + n2[None, :] - 2.0 * g
        mask = (1.0
                - (colid == lf_ref[...]).astype(_f32)
                - (colid == rf_ref[...]).astype(_f32))
        x = (posb[...] - sq + GAMMA) * mask
        x = jnp.where(valid, x, 0.0)
        X[:, pl.ds(j * NB, NB)] = x
        sumx[...] += jnp.sum(x, axis=1, keepdims=True)
        sumx2[...] += jnp.sum(x * x, axis=1, keepdims=True)
        xm = jnp.max(jnp.where(valid, x, -jnp.inf), axis=1, keepdims=True)
        xmax[...] = jnp.maximum(xmax[...], xm)

    @pl.when(p == 1)
    def _():
        m = sumx[...] * (1.0 / NODE)
        sd = jnp.sqrt(sumx2[...] * (1.0 / NODE) - m * m)
        zmax = 20.0 * (xmax[...] - m) / sd + 8.0
        x = X[:, pl.ds(j * NB, NB)]
        zz = 20.0 * (x - m) / sd + 8.0
        pe = jnp.where(valid, jnp.exp(zz - zmax), 0.0)
        sumexp[...] += jnp.sum(pe, axis=1, keepdims=True)

        @pl.when(j == NJ - 1)
        def _():
            ll = zmax + jnp.log(sumexp[...])
            out_ref[...] = jnp.reshape(jnp.sum(ll) * (1.0 / 256.0), (1, 1))


def _loss(A, lf, rf, emb):
    return pl.pallas_call(
        _loss_body,
        grid=(2, NJ),
        in_specs=[
            pl.BlockSpec((M, 6 * D), lambda p, j: (0, 0)),
            pl.BlockSpec((M, 1), lambda p, j: (0, 0)),
            pl.BlockSpec((M, 1), lambda p, j: (0, 0)),
            pl.BlockSpec((NB, 6 * D), lambda p, j: (j, 0)),
        ],
        out_specs=pl.BlockSpec((1, 1), lambda p, j: (0, 0)),
        out_shape=jax.ShapeDtypeStruct((1, 1), _f32),
        scratch_shapes=[
            pltpu.VMEM((M, NODEP), _f32),
            pltpu.VMEM((M, 1), _f32),
            pltpu.VMEM((M, 1), _f32),
            pltpu.VMEM((M, 1), _f32),
            pltpu.VMEM((M, 1), _f32),
            pltpu.VMEM((M, 1), _f32),
        ],
    )(A, lf, rf, emb)


def kernel(train_pairs, adj, r_index1, r_val, ent_adj, rel_adj_row,
           rel_adj_col, ent_table, rel_table, attn_e0, attn_e1, attn_r0,
           attn_r1):
    src = adj[0]
    dst = adj[1]
    i32 = jnp.int32
    nrm = jnp.sqrt(jnp.sum(rel_table * rel_table, axis=1))
    nrmc = jnp.maximum(nrm, 1e-30)
    U = rel_table / nrmc[:, None]

    def etab(ak):
        return jnp.exp((rel_table @ ak)[:, 0] / nrmc)

    Ea2 = jnp.concatenate([etab(attn_e0), etab(attn_r0)])
    Eb2 = jnp.concatenate([etab(attn_e1), etab(attn_r1)])

    row2 = jnp.concatenate([ent_adj[0], rel_adj_row])
    col2 = jnp.concatenate([ent_adj[1], rel_adj_col])
    tab2 = jnp.concatenate([ent_table, rel_table], axis=0)

    padS = T2S - T
    srcS = jnp.concatenate([src, jnp.full((padS,), NODE, i32)])
    relS = jnp.concatenate([r_index1, jnp.zeros((padS,), i32)])
    rvbS = jnp.concatenate([lax.bitcast_convert_type(r_val, i32),
                            jnp.full((padS,), 0x3F800000, i32)])
    rowS0 = jnp.concatenate([ent_adj[0], jnp.full((padS,), NODE, i32)])
    rowS1 = jnp.concatenate([rel_adj_row, jnp.full((padS,), NODE, i32)])

    def _ilv4(a, b, cc, dd, blk):
        return (jnp.stack([a, b, cc, dd], axis=0)
                .reshape(4, a.shape[0] // blk, blk)
                .transpose(1, 0, 2)
                .reshape(-1))

    sdata = jnp.concatenate([_ilv4(rowS0, srcS, relS, rvbS, SEB),
                             _ilv4(rowS1, srcS, relS, rvbS, SEB)])
    invc2, inv_sA2, inv_sB2 = _stats(sdata, Ea2, Eb2, nrm)

    pad = T2 - T
    srcp = jnp.concatenate([src, jnp.full((pad,), NODE, i32)])
    dstp = jnp.concatenate([dst, jnp.full((pad,), NODE, i32)])
    relp = jnp.concatenate([r_index1, jnp.zeros((pad,), i32)])
    rvb = lax.bitcast_convert_type(r_val, i32)
    rvbp = jnp.concatenate([rvb, jnp.full((pad,), 0x3F800000, i32)])
    edata = (jnp.stack([srcp, dstp, relp, rvbp], axis=0)
             .reshape(4, T2 // LEB, LEB)
             .transpose(1, 0, 2)
             .reshape(-1))

    padA = T2A - T
    rowp0 = jnp.concatenate([ent_adj[0], jnp.full((padA,), NODE, i32)])
    colp0 = jnp.concatenate([ent_adj[1], jnp.zeros((padA,), i32)])
    rowp1 = jnp.concatenate([rel_adj_row, jnp.full((padA,), NODE, i32)])
    colp1 = jnp.concatenate([rel_adj_col, jnp.zeros((padA,), i32)])

    def _ilv(rowp, colp):
        return (jnp.stack([rowp, colp], axis=0)
                .reshape(2, T2A // AEB, AEB)
                .transpose(1, 0, 2)
                .reshape(-1))

    adata = jnp.concatenate([_ilv(rowp0, colp0), _ilv(rowp1, colp1)])
    F0 = _avg(adata, tab2, invc2)
    F1 = _layer(edata, U, F0, Ea2, nrm, inv_sA2)
    F2 = _layer(edata, U, F1, Eb2, nrm, inv_sB2)

    emb = jnp.concatenate([F0[:NODEP], F1[:NODEP], F2[:NODEP],
                           F0[NODEP:], F1[NODEP:], F2[NODEP:]], axis=1)
    l = train_pairs[:, 0]
    r = train_pairs[:, 1]
    A = jnp.concatenate([emb[l], emb[r]], axis=0)
    lf = jnp.concatenate([l, r]).astype(_f32)[:, None]
    rf = jnp.concatenate([r, l]).astype(_f32)[:, None]
    loss = _loss(A, lf, rf, emb)
    return jnp.reshape(loss, ())
